# baseline (speedup 1.0000x reference)
import math

import jax
import jax.numpy as jnp
from jax import lax
from jax.experimental import pallas as pl
from jax.experimental.pallas import tpu as pltpu
from jax.experimental.pallas import tpu_sc as plsc

N = 10000
E = 320000
D = 128

NC = 2
NS = 16
NW = NC * NS
L = 16

CH = 128
ET = E + N
NCH = (ET + NW * CH - 1) // (NW * CH)
EPW = NCH * CH
EP = EPW * NW
SB = 8
NSB = (NCH + SB - 1) // SB
NCHP = NSB * SB
NP = 10112
RPT = NP // NS
ZR = 128


def _sc_agg_body(h_hbm, src_hbm, dst_hbm, w_hbm, out_hbm,
                 src_v, dst_v, w_v, rows_v, acc_sh, gsem, ssem, isem):
    c = lax.axis_index("c")
    s = lax.axis_index("s")
    wid = c * NS + s

    pltpu.sync_copy(src_hbm.at[wid, pl.ds(0, SB)], src_v.at[pl.ds(0, SB)])
    pltpu.sync_copy(dst_hbm.at[wid, pl.ds(0, SB)], dst_v.at[pl.ds(0, SB)])
    pltpu.sync_copy(w_hbm.at[wid, pl.ds(0, SB)], w_v.at[pl.ds(0, SB)])

    zvec = jnp.zeros((L,), jnp.float32)

    def zero_row(r, carry):
        for j in range(D // L):
            rows_v[0, r, pl.ds(j * L, L)] = zvec
        return carry

    lax.fori_loop(0, ZR, zero_row, 0)
    for k in range(RPT // ZR):
        pltpu.sync_copy(rows_v.at[0], acc_sh.at[pl.ds(s * RPT + k * ZR, ZR)])
    rem = RPT - (RPT // ZR) * ZR
    if rem:
        pltpu.sync_copy(rows_v.at[0, pl.ds(0, rem)],
                        acc_sh.at[pl.ds(s * RPT + (RPT // ZR) * ZR, rem)])
    plsc.subcore_barrier()

    pltpu.async_copy(h_hbm.at[src_v.at[0]], rows_v.at[0], gsem)

    def chunk(i, carry):
        sb = i // SB
        r = i - sb * SB
        b = lax.rem(i, 2)
        row = lax.rem(sb, 2) * SB + r

        pltpu.make_async_copy(h_hbm.at[src_v.at[row]], rows_v.at[b],
                              gsem).wait()

        @pl.when(jnp.logical_and(r == SB - 1, i < NCH - 1))
        def _stage_wait():
            nxt = sb + 1
            drow = lax.rem(nxt, 2) * SB
            pltpu.make_async_copy(src_hbm.at[wid, pl.ds(nxt * SB, SB)],
                                  src_v.at[pl.ds(drow, SB)], isem).wait()
            pltpu.make_async_copy(dst_hbm.at[wid, pl.ds(nxt * SB, SB)],
                                  dst_v.at[pl.ds(drow, SB)], isem).wait()
            pltpu.make_async_copy(w_hbm.at[wid, pl.ds(nxt * SB, SB)],
                                  w_v.at[pl.ds(drow, SB)], isem).wait()

        @pl.when(i < NCH - 1)
        def _issue_next_gather():
            i1 = i + 1
            sb1 = i1 // SB
            row1 = lax.rem(sb1, 2) * SB + (i1 - sb1 * SB)
            nb = lax.rem(i1, 2)

            @pl.when(i >= 1)
            def _drain_prev_scatter():
                pltpu.make_async_copy(rows_v.at[nb],
                                      acc_sh.at[dst_v.at[row]], ssem).wait()

            pltpu.async_copy(h_hbm.at[src_v.at[row1]], rows_v.at[nb], gsem)

        @pl.when(jnp.logical_and(r == 0, sb < NSB - 1))
        def _stage_next():
            nxt = sb + 1
            drow = lax.rem(nxt, 2) * SB
            pltpu.async_copy(src_hbm.at[wid, pl.ds(nxt * SB, SB)],
                             src_v.at[pl.ds(drow, SB)], isem)
            pltpu.async_copy(dst_hbm.at[wid, pl.ds(nxt * SB, SB)],
                             dst_v.at[pl.ds(drow, SB)], isem)
            pltpu.async_copy(w_hbm.at[wid, pl.ds(nxt * SB, SB)],
                             w_v.at[pl.ds(drow, SB)], isem)

        def group(g, cc):
            w16 = w_v[row, pl.ds(g * L, L)]
            for el in range(L):
                w = w16[el]
                e = g * L + el
                for j in range(D // L):
                    rows_v[b, e, pl.ds(j * L, L)] = (
                        rows_v[b, e, pl.ds(j * L, L)] * w)
            return cc

        lax.fori_loop(0, CH // L, group, 0)
        pltpu.async_copy(rows_v.at[b], acc_sh.at[dst_v.at[row]], ssem,
                         add=True)
        return carry

    lax.fori_loop(0, NCH, chunk, 0)
    pltpu.make_async_copy(rows_v.at[(NCH - 2) % 2],
                          acc_sh.at[dst_v.at[0]], ssem).wait()
    pltpu.make_async_copy(rows_v.at[(NCH - 1) % 2],
                          acc_sh.at[dst_v.at[0]], ssem).wait()
    plsc.subcore_barrier()

    for k in range(RPT // ZR):
        pltpu.sync_copy(acc_sh.at[pl.ds(s * RPT + k * ZR, ZR)], rows_v.at[0])
        pltpu.sync_copy(rows_v.at[0], out_hbm.at[c, pl.ds(s * RPT + k * ZR, ZR)])
    if RPT % ZR:
        k = RPT // ZR
        prem = RPT - k * ZR
        pltpu.sync_copy(acc_sh.at[pl.ds(s * RPT + k * ZR, prem)],
                        rows_v.at[0, pl.ds(0, prem)])
        pltpu.sync_copy(rows_v.at[0, pl.ds(0, prem)],
                        out_hbm.at[c, pl.ds(s * RPT + k * ZR, prem)])


_sc_agg = pl.kernel(
    _sc_agg_body,
    out_type=jax.ShapeDtypeStruct((NC, NP, D), jnp.float32),
    mesh=plsc.VectorSubcoreMesh(core_axis_name="c", subcore_axis_name="s"),
    scratch_types=[
        pltpu.VMEM((2 * SB, CH), jnp.int32),
        pltpu.VMEM((2 * SB, CH), jnp.int32),
        pltpu.VMEM((2 * SB, CH), jnp.float32),
        pltpu.VMEM((2, CH, D), jnp.float32),
        pltpu.VMEM_SHARED((NP, D), jnp.float32),
        pltpu.SemaphoreType.DMA,
        pltpu.SemaphoreType.DMA,
        pltpu.SemaphoreType.DMA,
    ],
)


def _mlp_body(p_ref, wt_ref, scale_ref, bias_ref, out_ref):
    comb = p_ref[0] + p_ref[1]
    y = jnp.dot(comb, wt_ref[...], preferred_element_type=jnp.float32)
    out_ref[...] = jnp.maximum(y * scale_ref[...] + bias_ref[...], 0.0)


def _mlp(p, wt, scale, bias):
    rb = 1000
    return pl.pallas_call(
        _mlp_body,
        grid=(N // rb,),
        in_specs=[
            pl.BlockSpec((NC, rb, D), lambda i: (0, i, 0)),
            pl.BlockSpec((D, D), lambda i: (0, 0)),
            pl.BlockSpec((1, D), lambda i: (0, 0)),
            pl.BlockSpec((1, D), lambda i: (0, 0)),
        ],
        out_specs=pl.BlockSpec((rb, D), lambda i: (i, 0)),
        out_shape=jax.ShapeDtypeStruct((N, D), jnp.float32),
    )(p, wt, scale, bias)


def kernel(x, edge_index, edge_weight, eps0, W1, b1, g1, be1,
           eps1, W2, b2, g2, be2, eps2, W3, b3):
    src = edge_index[0]
    dst = edge_index[1]
    node_ids = jnp.arange(N, dtype=jnp.int32)
    pad = EP - ET

    def lay3(parts):
        flat = jnp.concatenate(parts).reshape(NW, NCH, CH)
        tail = jnp.zeros((NW, NCHP - NCH, CH), flat.dtype)
        return jnp.concatenate([flat, tail], axis=1)

    src_p = lay3([src, node_ids, jnp.zeros((pad,), jnp.int32)])
    dst_p = lay3([dst, node_ids, jnp.zeros((pad,), jnp.int32)])

    def wts(eps):
        return lay3([
            edge_weight,
            jnp.broadcast_to(1.0 + eps[0], (N,)).astype(jnp.float32),
            jnp.zeros((pad,), jnp.float32),
        ])

    bn_s = jnp.float32(1.0 / math.sqrt(1.0 + 1e-5))
    scale1 = (g1 * bn_s).reshape(1, D)
    bias1 = (b1 * g1 * bn_s + be1).reshape(1, D)
    scale2 = (g2 * bn_s).reshape(1, D)
    bias2 = (b2 * g2 * bn_s + be2).reshape(1, D)
    scale3 = jnp.ones((1, D), jnp.float32)
    bias3 = b3.reshape(1, D)

    h = x
    for (wcat, W, sc, bi) in ((wts(eps0), W1, scale1, bias1),
                              (wts(eps1), W2, scale2, bias2),
                              (wts(eps2), W3, scale3, bias3)):
        p = _sc_agg(h, src_p, dst_p, wcat)
        h = _mlp(p, W.T, sc, bi)
    return h


# speedup vs baseline: 2.0347x; 2.0347x over previous
import math

import jax
import jax.numpy as jnp
from jax import lax
from jax.experimental import pallas as pl
from jax.experimental.pallas import tpu as pltpu
from jax.experimental.pallas import tpu_sc as plsc

N = 10000
E = 320000
D = 128

NC = 2
NS = 16
NW = NC * NS
L = 16

CH = 128
ET = E + N
NCH = (ET + NW * CH - 1) // (NW * CH)
EPW = NCH * CH
EP = EPW * NW
SB = 8
NSB = (NCH + SB - 1) // SB
NCHP = NSB * SB
NFB = NCH // SB
NP = 10112
RPT = NP // NS
ZR = 128


def _sc_agg_body(h_hbm, src_hbm, dst_hbm, w_hbm, out_hbm,
                 src_v, dst_v, w_v, rows_v, acc_sh, gsem, ssem, isem):
    c = lax.axis_index("c")
    s = lax.axis_index("s")
    wid = c * NS + s

    pltpu.sync_copy(src_hbm.at[wid, pl.ds(0, SB)], src_v.at[pl.ds(0, SB)])
    pltpu.sync_copy(dst_hbm.at[wid, pl.ds(0, SB)], dst_v.at[pl.ds(0, SB)])
    pltpu.sync_copy(w_hbm.at[wid, pl.ds(0, SB)], w_v.at[pl.ds(0, SB)])

    zvec = jnp.zeros((L,), jnp.float32)

    def zero_row(r, carry):
        for bb in range(2):
            for j in range(D // L):
                rows_v[bb, r, pl.ds(j * L, L)] = zvec
        return carry

    lax.fori_loop(0, ZR, zero_row, 0)
    for k in range(RPT // ZR):
        pltpu.sync_copy(rows_v.at[0], acc_sh.at[pl.ds(s * RPT + k * ZR, ZR)])
    rem = RPT - (RPT // ZR) * ZR
    if rem:
        pltpu.sync_copy(rows_v.at[0, pl.ds(0, rem)],
                        acc_sh.at[pl.ds(s * RPT + (RPT // ZR) * ZR, rem)])
    plsc.subcore_barrier()

    pltpu.async_copy(h_hbm.at[src_v.at[0]], rows_v.at[0], gsem)
    pltpu.async_copy(rows_v.at[1], acc_sh.at[dst_v.at[0]], ssem, add=True)

    def do_chunk(i, b, row, row1):
        nb = 1 - b
        pltpu.make_async_copy(h_hbm.at[src_v.at[row]], rows_v.at[b],
                              gsem).wait()
        pltpu.make_async_copy(rows_v.at[nb], acc_sh.at[dst_v.at[row]],
                              ssem).wait()
        pltpu.async_copy(h_hbm.at[src_v.at[row1]], rows_v.at[nb], gsem)

        def group(g, cc):
            w16 = w_v[row, pl.ds(g * L, L)]
            for el in range(L):
                w = w16[el]
                e = g * L + el
                for j in range(D // L):
                    rows_v[b, e, pl.ds(j * L, L)] = (
                        rows_v[b, e, pl.ds(j * L, L)] * w)
            return cc

        lax.fori_loop(0, CH // L, group, 0)
        pltpu.async_copy(rows_v.at[b], acc_sh.at[dst_v.at[row]], ssem,
                         add=True)

    def block(sb, carry):
        base = sb * SB
        cur = lax.rem(sb, 2) * SB
        nxtrow = lax.rem(sb + 1, 2) * SB
        for r in range(SB):
            i = base + r
            b = lax.rem(i, 2)
            row = cur + r
            row1 = nxtrow if r == SB - 1 else cur + r + 1
            if r == SB - 1:
                nxt = sb + 1
                pltpu.make_async_copy(src_hbm.at[wid, pl.ds(nxt * SB, SB)],
                                      src_v.at[pl.ds(nxtrow, SB)],
                                      isem).wait()
                pltpu.make_async_copy(dst_hbm.at[wid, pl.ds(nxt * SB, SB)],
                                      dst_v.at[pl.ds(nxtrow, SB)],
                                      isem).wait()
                pltpu.make_async_copy(w_hbm.at[wid, pl.ds(nxt * SB, SB)],
                                      w_v.at[pl.ds(nxtrow, SB)],
                                      isem).wait()
            do_chunk(i, b, row, row1)
            if r == 0:
                nxt = sb + 1
                pltpu.async_copy(src_hbm.at[wid, pl.ds(nxt * SB, SB)],
                                 src_v.at[pl.ds(nxtrow, SB)], isem)
                pltpu.async_copy(dst_hbm.at[wid, pl.ds(nxt * SB, SB)],
                                 dst_v.at[pl.ds(nxtrow, SB)], isem)
                pltpu.async_copy(w_hbm.at[wid, pl.ds(nxt * SB, SB)],
                                 w_v.at[pl.ds(nxtrow, SB)], isem)
        return carry

    lax.fori_loop(0, NFB, block, 0)

    lastb = (NCH - 1) % 2
    lastrow = (NFB % 2) * SB
    pltpu.make_async_copy(h_hbm.at[src_v.at[lastrow]], rows_v.at[lastb],
                          gsem).wait()
    pltpu.make_async_copy(rows_v.at[1 - lastb], acc_sh.at[dst_v.at[0]],
                          ssem).wait()

    def lgroup(g, cc):
        w16 = w_v[lastrow, pl.ds(g * L, L)]
        for el in range(L):
            w = w16[el]
            e = g * L + el
            for j in range(D // L):
                rows_v[lastb, e, pl.ds(j * L, L)] = (
                    rows_v[lastb, e, pl.ds(j * L, L)] * w)
        return cc

    lax.fori_loop(0, CH // L, lgroup, 0)
    pltpu.async_copy(rows_v.at[lastb], acc_sh.at[dst_v.at[lastrow]], ssem,
                     add=True)
    pltpu.make_async_copy(rows_v.at[lastb], acc_sh.at[dst_v.at[0]],
                          ssem).wait()
    plsc.subcore_barrier()

    for k in range(RPT // ZR):
        pltpu.sync_copy(acc_sh.at[pl.ds(s * RPT + k * ZR, ZR)], rows_v.at[0])
        pltpu.sync_copy(rows_v.at[0], out_hbm.at[c, pl.ds(s * RPT + k * ZR, ZR)])
    if RPT % ZR:
        k = RPT // ZR
        prem = RPT - k * ZR
        pltpu.sync_copy(acc_sh.at[pl.ds(s * RPT + k * ZR, prem)],
                        rows_v.at[0, pl.ds(0, prem)])
        pltpu.sync_copy(rows_v.at[0, pl.ds(0, prem)],
                        out_hbm.at[c, pl.ds(s * RPT + k * ZR, prem)])


_sc_agg = pl.kernel(
    _sc_agg_body,
    out_type=jax.ShapeDtypeStruct((NC, NP, D), jnp.float32),
    mesh=plsc.VectorSubcoreMesh(core_axis_name="c", subcore_axis_name="s"),
    scratch_types=[
        pltpu.VMEM((2 * SB, CH), jnp.int32),
        pltpu.VMEM((2 * SB, CH), jnp.int32),
        pltpu.VMEM((2 * SB, CH), jnp.float32),
        pltpu.VMEM((2, CH, D), jnp.float32),
        pltpu.VMEM_SHARED((NP, D), jnp.float32),
        pltpu.SemaphoreType.DMA,
        pltpu.SemaphoreType.DMA,
        pltpu.SemaphoreType.DMA,
    ],
)


def _mlp_body(p_ref, wt_ref, scale_ref, bias_ref, out_ref):
    comb = p_ref[0] + p_ref[1]
    y = jnp.dot(comb, wt_ref[...], preferred_element_type=jnp.float32)
    out_ref[...] = jnp.maximum(y * scale_ref[...] + bias_ref[...], 0.0)


def _mlp(p, wt, scale, bias):
    rb = 1000
    return pl.pallas_call(
        _mlp_body,
        grid=(N // rb,),
        in_specs=[
            pl.BlockSpec((NC, rb, D), lambda i: (0, i, 0)),
            pl.BlockSpec((D, D), lambda i: (0, 0)),
            pl.BlockSpec((1, D), lambda i: (0, 0)),
            pl.BlockSpec((1, D), lambda i: (0, 0)),
        ],
        out_specs=pl.BlockSpec((rb, D), lambda i: (i, 0)),
        out_shape=jax.ShapeDtypeStruct((N, D), jnp.float32),
    )(p, wt, scale, bias)


def kernel(x, edge_index, edge_weight, eps0, W1, b1, g1, be1,
           eps1, W2, b2, g2, be2, eps2, W3, b3):
    src = edge_index[0]
    dst = edge_index[1]
    node_ids = jnp.arange(N, dtype=jnp.int32)
    pad = EP - ET

    def lay3(parts):
        flat = jnp.concatenate(parts).reshape(NW, NCH, CH)
        tail = jnp.zeros((NW, NCHP - NCH, CH), flat.dtype)
        return jnp.concatenate([flat, tail], axis=1)

    src_p = lay3([src, node_ids, jnp.zeros((pad,), jnp.int32)])
    dst_p = lay3([dst, node_ids, jnp.zeros((pad,), jnp.int32)])

    def wts(eps):
        return lay3([
            edge_weight,
            jnp.broadcast_to(1.0 + eps[0], (N,)).astype(jnp.float32),
            jnp.zeros((pad,), jnp.float32),
        ])

    bn_s = jnp.float32(1.0 / math.sqrt(1.0 + 1e-5))
    scale1 = (g1 * bn_s).reshape(1, D)
    bias1 = (b1 * g1 * bn_s + be1).reshape(1, D)
    scale2 = (g2 * bn_s).reshape(1, D)
    bias2 = (b2 * g2 * bn_s + be2).reshape(1, D)
    scale3 = jnp.ones((1, D), jnp.float32)
    bias3 = b3.reshape(1, D)

    h = x
    for (wcat, W, sc, bi) in ((wts(eps0), W1, scale1, bias1),
                              (wts(eps1), W2, scale2, bias2),
                              (wts(eps2), W3, scale3, bias3)):
        p = _sc_agg(h, src_p, dst_p, wcat)
        h = _mlp(p, W.T, sc, bi)
    return h
